# Initial kernel scaffold; baseline (speedup 1.0000x reference)
import jax
import jax.numpy as jnp
from jax.experimental import pallas as pl


def kernel(x, edge_index, edge_weight, W, b):
    raise NotImplementedError("write your pallas kernel here")


# speedup vs baseline: 5.1055x; 5.1055x over previous
import functools

import jax
import jax.numpy as jnp
from jax import lax
from jax.experimental import pallas as pl
from jax.experimental.pallas import tpu as pltpu
from jax.experimental.pallas import tpu_sc as plsc

N_NODES = 10000
N_EDGES = 320000
D = 128
NC = 2
NS = 16
NW = NC * NS
VEC = 16

CHUNK = 80
EPW = N_EDGES // NW
NCHUNK = EPW // CHUNK
ROWS_PER_TILE = 624
TAIL_ROW0 = NS * ROWS_PER_TILE
TAIL_ROWS = N_NODES - TAIL_ROW0
NFULL = ROWS_PER_TILE // CHUNK
REM = ROWS_PER_TILE - NFULL * CHUNK


def _sc_aggregate(x, src_r, dst_r, w_r):
    mesh = plsc.VectorSubcoreMesh(
        core_axis_name="c", subcore_axis_name="s",
        num_cores=NC, num_subcores=NS)

    @functools.partial(
        pl.kernel,
        out_type=jax.ShapeDtypeStruct((NC, N_NODES, D), jnp.float32),
        mesh=mesh,
        scratch_types=[
            pltpu.VMEM((CHUNK,), jnp.int32),
            pltpu.VMEM((CHUNK,), jnp.int32),
            pltpu.VMEM((NCHUNK, CHUNK), jnp.float32),
            pltpu.VMEM((CHUNK, D), jnp.float32),
            pltpu.VMEM_SHARED((N_NODES, D), jnp.float32),
            pltpu.SemaphoreType.DMA,
        ],
    )
    def agg(x_hbm, src_hbm, dst_hbm, w_hbm, part_hbm,
            src_c, dst_c, w_v, rows_v, acc_sh, sem):
        cid = lax.axis_index("c")
        sid = lax.axis_index("s")
        wid = cid * NS + sid

        pltpu.sync_copy(w_hbm.at[wid], w_v)

        def zrow(i, c):
            for j in range(D // VEC):
                rows_v[i, pl.ds(j * VEC, VEC)] = jnp.zeros((VEC,), jnp.float32)
            return c
        lax.fori_loop(0, CHUNK, zrow, 0)
        base = sid * ROWS_PER_TILE
        for i in range(NFULL):
            pltpu.sync_copy(rows_v, acc_sh.at[pl.ds(base + i * CHUNK, CHUNK)])
        pltpu.sync_copy(rows_v.at[pl.ds(0, REM)],
                        acc_sh.at[pl.ds(base + NFULL * CHUNK, REM)])

        @pl.when(sid == NS - 1)
        def _zero_tail():
            pltpu.sync_copy(rows_v.at[pl.ds(0, TAIL_ROWS)],
                            acc_sh.at[pl.ds(TAIL_ROW0, TAIL_ROWS)])
        plsc.subcore_barrier()

        def chunk_body(t, c):
            e0 = wid * EPW + t * CHUNK
            pltpu.sync_copy(src_hbm.at[pl.ds(e0, CHUNK)], src_c)
            pltpu.sync_copy(dst_hbm.at[pl.ds(e0, CHUNK)], dst_c)
            pltpu.async_copy(x_hbm.at[src_c], rows_v, sem).wait()

            def grp_body(g, c2):
                w16 = w_v[t, pl.ds(g * VEC, VEC)]
                for e in range(VEC):
                    wbc = jnp.full((VEC,), w16[e], jnp.float32)
                    row = g * VEC + e
                    for j in range(D // VEC):
                        sl = pl.ds(j * VEC, VEC)
                        rows_v[row, sl] = rows_v[row, sl] * wbc
                return c2
            lax.fori_loop(0, CHUNK // VEC, grp_body, 0)

            pltpu.sync_copy(rows_v, acc_sh.at[dst_c], add=True)
            return c
        lax.fori_loop(0, NCHUNK, chunk_body, 0)
        plsc.subcore_barrier()

        for i in range(NFULL):
            off = base + i * CHUNK
            pltpu.sync_copy(acc_sh.at[pl.ds(off, CHUNK)], rows_v)
            pltpu.sync_copy(rows_v, part_hbm.at[cid, pl.ds(off, CHUNK)])
        off = base + NFULL * CHUNK
        pltpu.sync_copy(acc_sh.at[pl.ds(off, REM)], rows_v.at[pl.ds(0, REM)])
        pltpu.sync_copy(rows_v.at[pl.ds(0, REM)], part_hbm.at[cid, pl.ds(off, REM)])

        @pl.when(sid == NS - 1)
        def _dump_tail():
            pltpu.sync_copy(acc_sh.at[pl.ds(TAIL_ROW0, TAIL_ROWS)],
                            rows_v.at[pl.ds(0, TAIL_ROWS)])
            pltpu.sync_copy(rows_v.at[pl.ds(0, TAIL_ROWS)],
                            part_hbm.at[cid, pl.ds(TAIL_ROW0, TAIL_ROWS)])

    return agg(x, src_r, dst_r, w_r)


def _tc_combine_mm(parts, W, b):
    def body(p_ref, w_ref, b_ref, o_ref):
        acc = p_ref[0] + p_ref[1]
        o_ref[...] = jnp.dot(acc, w_ref[...],
                             preferred_element_type=jnp.float32) + b_ref[...]

    BM = 1000
    return pl.pallas_call(
        body,
        grid=(N_NODES // BM,),
        in_specs=[
            pl.BlockSpec((NC, BM, D), lambda i: (0, i, 0)),
            pl.BlockSpec((D, D), lambda i: (0, 0)),
            pl.BlockSpec((1, D), lambda i: (0, 0)),
        ],
        out_specs=pl.BlockSpec((BM, D), lambda i: (i, 0)),
        out_shape=jax.ShapeDtypeStruct((N_NODES, D), jnp.float32),
    )(parts, W, b.reshape(1, D))


def kernel(x, edge_index, edge_weight, W, b):
    src = edge_index[0].astype(jnp.int32)
    dst = edge_index[1].astype(jnp.int32)
    w = edge_weight.astype(jnp.float32).reshape(NW, NCHUNK, CHUNK)
    parts = _sc_aggregate(x, src, dst, w)
    return _tc_combine_mm(parts, W, b)


# speedup vs baseline: 8.5556x; 1.6758x over previous
import functools

import jax
import jax.numpy as jnp
from jax import lax
from jax.experimental import pallas as pl
from jax.experimental.pallas import tpu as pltpu
from jax.experimental.pallas import tpu_sc as plsc

N_NODES = 10000
N_EDGES = 320000
D = 128
NC = 2
NS = 16
NW = NC * NS
VEC = 16

CHUNK = 80
EPW = N_EDGES // NW
NCHUNK = EPW // CHUNK
ROWS_PER_TILE = 624
TAIL_ROW0 = NS * ROWS_PER_TILE
TAIL_ROWS = N_NODES - TAIL_ROW0
NFULL = ROWS_PER_TILE // CHUNK
REM = ROWS_PER_TILE - NFULL * CHUNK


def _sc_aggregate(x, sdw):
    mesh = plsc.VectorSubcoreMesh(
        core_axis_name="c", subcore_axis_name="s",
        num_cores=NC, num_subcores=NS)

    @functools.partial(
        pl.kernel,
        out_type=jax.ShapeDtypeStruct((NC, N_NODES, D), jnp.float32),
        mesh=mesh,
        scratch_types=[
            pltpu.VMEM((3, CHUNK), jnp.int32),
            pltpu.VMEM((3, CHUNK), jnp.int32),
            pltpu.VMEM((CHUNK, D), jnp.float32),
            pltpu.VMEM((CHUNK, D), jnp.float32),
            pltpu.VMEM_SHARED((N_NODES, D), jnp.float32),
            pltpu.SemaphoreType.DMA,
            pltpu.SemaphoreType.DMA,
            pltpu.SemaphoreType.DMA,
            pltpu.SemaphoreType.DMA,
        ],
    )
    def agg(x_hbm, sdw_hbm, part_hbm,
            sdw0, sdw1, rows0, rows1, acc_sh, gsem0, gsem1, isem0, isem1):
        cid = lax.axis_index("c")
        sid = lax.axis_index("s")
        wid = cid * NS + sid
        bt = wid * NCHUNK

        pltpu.sync_copy(sdw_hbm.at[bt], sdw0)
        pltpu.async_copy(x_hbm.at[sdw0.at[0]], rows0, gsem0)
        pltpu.async_copy(sdw_hbm.at[bt + 1], sdw1, isem1)

        def zrow(i, c):
            for j in range(D // VEC):
                rows1[i, pl.ds(j * VEC, VEC)] = jnp.zeros((VEC,), jnp.float32)
            return c
        lax.fori_loop(0, CHUNK, zrow, 0)
        base = sid * ROWS_PER_TILE
        for i in range(NFULL):
            pltpu.sync_copy(rows1, acc_sh.at[pl.ds(base + i * CHUNK, CHUNK)])
        pltpu.sync_copy(rows1.at[pl.ds(0, REM)],
                        acc_sh.at[pl.ds(base + NFULL * CHUNK, REM)])

        @pl.when(sid == NS - 1)
        def _zero_tail():
            pltpu.sync_copy(rows1.at[pl.ds(0, TAIL_ROWS)],
                            acc_sh.at[pl.ds(TAIL_ROW0, TAIL_ROWS)])
        plsc.subcore_barrier()

        def one_iter(t, sdw_p, rows_p, gsem_p, sdw_q, rows_q, gsem_q,
                     isem_p, isem_q):
            @pl.when(t + 1 < NCHUNK)
            def _launch_next():
                pltpu.make_async_copy(sdw_hbm.at[bt + t + 1], sdw_q,
                                      isem_q).wait()
                pltpu.async_copy(x_hbm.at[sdw_q.at[0]], rows_q, gsem_q)

            pltpu.make_async_copy(x_hbm.at[sdw_p.at[0]], rows_p, gsem_p).wait()

            def grp_body(g, c):
                w16 = lax.bitcast_convert_type(
                    sdw_p[2, pl.ds(g * VEC, VEC)], jnp.float32)
                for e in range(VEC):
                    wbc = jnp.full((VEC,), w16[e], jnp.float32)
                    row = g * VEC + e
                    for j in range(D // VEC):
                        sl = pl.ds(j * VEC, VEC)
                        rows_p[row, sl] = rows_p[row, sl] * wbc
                return c
            lax.fori_loop(0, CHUNK // VEC, grp_body, 0)

            pltpu.sync_copy(rows_p, acc_sh.at[sdw_p.at[1]], add=True)

            @pl.when(t + 2 < NCHUNK)
            def _prefetch_rec():
                pltpu.async_copy(sdw_hbm.at[bt + t + 2], sdw_p, isem_p)

        def body(t, c):
            @pl.when(lax.rem(t, 2) == 0)
            def _even():
                one_iter(t, sdw0, rows0, gsem0, sdw1, rows1, gsem1,
                         isem0, isem1)

            @pl.when(lax.rem(t, 2) == 1)
            def _odd():
                one_iter(t, sdw1, rows1, gsem1, sdw0, rows0, gsem0,
                         isem1, isem0)
            return c
        lax.fori_loop(0, NCHUNK, body, 0)
        plsc.subcore_barrier()

        for i in range(NFULL):
            off = base + i * CHUNK
            pltpu.sync_copy(acc_sh.at[pl.ds(off, CHUNK)], rows0)
            pltpu.sync_copy(rows0, part_hbm.at[cid, pl.ds(off, CHUNK)])
        off = base + NFULL * CHUNK
        pltpu.sync_copy(acc_sh.at[pl.ds(off, REM)], rows0.at[pl.ds(0, REM)])
        pltpu.sync_copy(rows0.at[pl.ds(0, REM)], part_hbm.at[cid, pl.ds(off, REM)])

        @pl.when(sid == NS - 1)
        def _dump_tail():
            pltpu.sync_copy(acc_sh.at[pl.ds(TAIL_ROW0, TAIL_ROWS)],
                            rows1.at[pl.ds(0, TAIL_ROWS)])
            pltpu.sync_copy(rows1.at[pl.ds(0, TAIL_ROWS)],
                            part_hbm.at[cid, pl.ds(TAIL_ROW0, TAIL_ROWS)])

    return agg(x, sdw)


def _tc_combine_mm(parts, W, b):
    def body(p_ref, w_ref, b_ref, o_ref):
        acc = p_ref[0] + p_ref[1]
        o_ref[...] = jnp.dot(acc, w_ref[...],
                             preferred_element_type=jnp.float32) + b_ref[...]

    BM = 1000
    return pl.pallas_call(
        body,
        grid=(N_NODES // BM,),
        in_specs=[
            pl.BlockSpec((NC, BM, D), lambda i: (0, i, 0)),
            pl.BlockSpec((D, D), lambda i: (0, 0)),
            pl.BlockSpec((1, D), lambda i: (0, 0)),
        ],
        out_specs=pl.BlockSpec((BM, D), lambda i: (i, 0)),
        out_shape=jax.ShapeDtypeStruct((N_NODES, D), jnp.float32),
    )(parts, W, b.reshape(1, D))


def kernel(x, edge_index, edge_weight, W, b):
    src = edge_index[0].astype(jnp.int32).reshape(-1, CHUNK)
    dst = edge_index[1].astype(jnp.int32).reshape(-1, CHUNK)
    w_i = lax.bitcast_convert_type(
        edge_weight.astype(jnp.float32), jnp.int32).reshape(-1, CHUNK)
    sdw = jnp.stack([src, dst, w_i], axis=1)
    parts = _sc_aggregate(x, sdw)
    return _tc_combine_mm(parts, W, b)


# speedup vs baseline: 10.0521x; 1.1749x over previous
import functools

import jax
import jax.numpy as jnp
from jax import lax
from jax.experimental import pallas as pl
from jax.experimental.pallas import tpu as pltpu
from jax.experimental.pallas import tpu_sc as plsc

N_NODES = 10000
N_EDGES = 320000
D = 128
NC = 2
NS = 16
NW = NC * NS
VEC = 16

CHUNK = 80
HALF_A = 48
HALF_B = CHUNK - HALF_A
EPW = N_EDGES // NW
NCHUNK = EPW // CHUNK
ROWS_PER_TILE = 624
TAIL_ROW0 = NS * ROWS_PER_TILE
TAIL_ROWS = N_NODES - TAIL_ROW0
NFULL = ROWS_PER_TILE // CHUNK
REM = ROWS_PER_TILE - NFULL * CHUNK


def _sc_aggregate(x, sdw):
    mesh = plsc.VectorSubcoreMesh(
        core_axis_name="c", subcore_axis_name="s",
        num_cores=NC, num_subcores=NS)

    @functools.partial(
        pl.kernel,
        out_type=jax.ShapeDtypeStruct((NC, N_NODES, D), jnp.float32),
        mesh=mesh,
        scratch_types=[
            pltpu.VMEM((3, CHUNK), jnp.int32),
            pltpu.VMEM((3, CHUNK), jnp.int32),
            pltpu.VMEM((CHUNK, D), jnp.float32),
            pltpu.VMEM((CHUNK, D), jnp.float32),
            pltpu.VMEM((HALF_A,), jnp.int32),
            pltpu.VMEM((HALF_A,), jnp.int32),
            pltpu.VMEM((HALF_B,), jnp.int32),
            pltpu.VMEM((HALF_B,), jnp.int32),
            pltpu.VMEM_SHARED((N_NODES, D), jnp.float32),
            pltpu.SemaphoreType.DMA,
            pltpu.SemaphoreType.DMA,
            pltpu.SemaphoreType.DMA,
            pltpu.SemaphoreType.DMA,
            pltpu.SemaphoreType.DMA,
            pltpu.SemaphoreType.DMA,
        ],
    )
    def agg(x_hbm, sdw_hbm, part_hbm,
            sdw0, sdw1, rows0, rows1, dhA0, dhA1, dhB0, dhB1, acc_sh,
            gsem0, gsem1, isem0, isem1, ssem0, ssem1):
        cid = lax.axis_index("c")
        sid = lax.axis_index("s")
        wid = cid * NS + sid
        bt = wid * NCHUNK

        pltpu.sync_copy(sdw_hbm.at[bt], sdw0)
        pltpu.async_copy(x_hbm.at[sdw0.at[0]], rows0, gsem0)
        pltpu.async_copy(sdw_hbm.at[bt + 1], sdw1, isem1)

        def zrow(i, c):
            for j in range(D // VEC):
                rows1[i, pl.ds(j * VEC, VEC)] = jnp.zeros((VEC,), jnp.float32)
            return c
        lax.fori_loop(0, CHUNK, zrow, 0)
        base = sid * ROWS_PER_TILE
        for i in range(NFULL):
            pltpu.sync_copy(rows1, acc_sh.at[pl.ds(base + i * CHUNK, CHUNK)])
        pltpu.sync_copy(rows1.at[pl.ds(0, REM)],
                        acc_sh.at[pl.ds(base + NFULL * CHUNK, REM)])

        @pl.when(sid == NS - 1)
        def _zero_tail():
            pltpu.sync_copy(rows1.at[pl.ds(0, TAIL_ROWS)],
                            acc_sh.at[pl.ds(TAIL_ROW0, TAIL_ROWS)])
        plsc.subcore_barrier()

        def drain_scatters(rows_r, dhA_r, dhB_r, ssem_r):
            pltpu.make_async_copy(rows_r.at[pl.ds(0, HALF_A)],
                                  acc_sh.at[dhA_r], ssem_r).wait()
            pltpu.make_async_copy(rows_r.at[pl.ds(HALF_A, HALF_B)],
                                  acc_sh.at[dhB_r], ssem_r).wait()

        def one_iter(t, sdw_p, rows_p, gsem_p, dhA_p, dhB_p, ssem_p,
                     sdw_q, rows_q, gsem_q, dhA_q, dhB_q, ssem_q,
                     isem_p, isem_q):
            @pl.when(t + 1 < NCHUNK)
            def _launch_next():
                pltpu.make_async_copy(sdw_hbm.at[bt + t + 1], sdw_q,
                                      isem_q).wait()

                @pl.when(t >= 1)
                def _drain_prev():
                    drain_scatters(rows_q, dhA_q, dhB_q, ssem_q)
                pltpu.async_copy(x_hbm.at[sdw_q.at[0]], rows_q, gsem_q)

            pltpu.make_async_copy(x_hbm.at[sdw_p.at[0]], rows_p, gsem_p).wait()

            for h in range(HALF_A // VEC):
                dhA_p[pl.ds(h * VEC, VEC)] = sdw_p[1, pl.ds(h * VEC, VEC)]
            for h in range(HALF_B // VEC):
                dhB_p[pl.ds(h * VEC, VEC)] = sdw_p[
                    1, pl.ds(HALF_A + h * VEC, VEC)]

            def grp_body(g, c):
                w16 = lax.bitcast_convert_type(
                    sdw_p[2, pl.ds(g * VEC, VEC)], jnp.float32)
                for e in range(VEC):
                    wbc = jnp.full((VEC,), w16[e], jnp.float32)
                    row = g * VEC + e
                    for j in range(D // VEC):
                        sl = pl.ds(j * VEC, VEC)
                        rows_p[row, sl] = rows_p[row, sl] * wbc
                return c
            lax.fori_loop(0, HALF_A // VEC, grp_body, 0)
            pltpu.async_copy(rows_p.at[pl.ds(0, HALF_A)],
                             acc_sh.at[dhA_p], ssem_p, add=True)
            lax.fori_loop(HALF_A // VEC, CHUNK // VEC, grp_body, 0)
            pltpu.async_copy(rows_p.at[pl.ds(HALF_A, HALF_B)],
                             acc_sh.at[dhB_p], ssem_p, add=True)

            @pl.when(t + 2 < NCHUNK)
            def _prefetch_rec():
                pltpu.async_copy(sdw_hbm.at[bt + t + 2], sdw_p, isem_p)

        def body(t, c):
            @pl.when(lax.rem(t, 2) == 0)
            def _even():
                one_iter(t, sdw0, rows0, gsem0, dhA0, dhB0, ssem0,
                         sdw1, rows1, gsem1, dhA1, dhB1, ssem1,
                         isem0, isem1)

            @pl.when(lax.rem(t, 2) == 1)
            def _odd():
                one_iter(t, sdw1, rows1, gsem1, dhA1, dhB1, ssem1,
                         sdw0, rows0, gsem0, dhA0, dhB0, ssem0,
                         isem1, isem0)
            return c
        lax.fori_loop(0, NCHUNK, body, 0)
        drain_scatters(rows1, dhA1, dhB1, ssem1)
        drain_scatters(rows0, dhA0, dhB0, ssem0)
        plsc.subcore_barrier()

        for i in range(NFULL):
            off = base + i * CHUNK
            pltpu.sync_copy(acc_sh.at[pl.ds(off, CHUNK)], rows0)
            pltpu.sync_copy(rows0, part_hbm.at[cid, pl.ds(off, CHUNK)])
        off = base + NFULL * CHUNK
        pltpu.sync_copy(acc_sh.at[pl.ds(off, REM)], rows0.at[pl.ds(0, REM)])
        pltpu.sync_copy(rows0.at[pl.ds(0, REM)], part_hbm.at[cid, pl.ds(off, REM)])

        @pl.when(sid == NS - 1)
        def _dump_tail():
            pltpu.sync_copy(acc_sh.at[pl.ds(TAIL_ROW0, TAIL_ROWS)],
                            rows1.at[pl.ds(0, TAIL_ROWS)])
            pltpu.sync_copy(rows1.at[pl.ds(0, TAIL_ROWS)],
                            part_hbm.at[cid, pl.ds(TAIL_ROW0, TAIL_ROWS)])

    return agg(x, sdw)


def _tc_combine_mm(parts, W, b):
    def body(p_ref, w_ref, b_ref, o_ref):
        acc = p_ref[0] + p_ref[1]
        o_ref[...] = jnp.dot(acc, w_ref[...],
                             preferred_element_type=jnp.float32) + b_ref[...]

    BM = 1000
    return pl.pallas_call(
        body,
        grid=(N_NODES // BM,),
        in_specs=[
            pl.BlockSpec((NC, BM, D), lambda i: (0, i, 0)),
            pl.BlockSpec((D, D), lambda i: (0, 0)),
            pl.BlockSpec((1, D), lambda i: (0, 0)),
        ],
        out_specs=pl.BlockSpec((BM, D), lambda i: (i, 0)),
        out_shape=jax.ShapeDtypeStruct((N_NODES, D), jnp.float32),
    )(parts, W, b.reshape(1, D))


def kernel(x, edge_index, edge_weight, W, b):
    src = edge_index[0].astype(jnp.int32).reshape(-1, CHUNK)
    dst = edge_index[1].astype(jnp.int32).reshape(-1, CHUNK)
    w_i = lax.bitcast_convert_type(
        edge_weight.astype(jnp.float32), jnp.int32).reshape(-1, CHUNK)
    sdw = jnp.stack([src, dst, w_i], axis=1)
    parts = _sc_aggregate(x, sdw)
    return _tc_combine_mm(parts, W, b)


# speedup vs baseline: 11.6399x; 1.1580x over previous
import functools

import jax
import jax.numpy as jnp
from jax import lax
from jax.experimental import pallas as pl
from jax.experimental.pallas import tpu as pltpu
from jax.experimental.pallas import tpu_sc as plsc

N_NODES = 10000
N_EDGES = 320000
D = 128
NC = 2
NS = 16
NW = NC * NS
VEC = 16

CHUNK = 80
HALF_A = 48
HALF_B = CHUNK - HALF_A
EPW = N_EDGES // NW
NCHUNK = EPW // CHUNK
ROWS_PER_TILE = 624
TAIL_ROW0 = NS * ROWS_PER_TILE
TAIL_ROWS = N_NODES - TAIL_ROW0
NFULL = ROWS_PER_TILE // CHUNK
REM = ROWS_PER_TILE - NFULL * CHUNK


def _sc_aggregate(x, ei_flat, ew):
    mesh = plsc.VectorSubcoreMesh(
        core_axis_name="c", subcore_axis_name="s",
        num_cores=NC, num_subcores=NS)

    @functools.partial(
        pl.kernel,
        out_type=jax.ShapeDtypeStruct((NC, N_NODES, D), jnp.float32),
        mesh=mesh,
        scratch_types=[
            pltpu.VMEM((CHUNK,), jnp.int32),
            pltpu.VMEM((CHUNK,), jnp.int32),
            pltpu.VMEM((CHUNK,), jnp.int32),
            pltpu.VMEM((CHUNK,), jnp.int32),
            pltpu.VMEM((CHUNK,), jnp.float32),
            pltpu.VMEM((CHUNK,), jnp.float32),
            pltpu.VMEM((CHUNK, D), jnp.float32),
            pltpu.VMEM((CHUNK, D), jnp.float32),
            pltpu.VMEM((HALF_A,), jnp.int32),
            pltpu.VMEM((HALF_A,), jnp.int32),
            pltpu.VMEM((HALF_B,), jnp.int32),
            pltpu.VMEM((HALF_B,), jnp.int32),
            pltpu.VMEM_SHARED((N_NODES, D), jnp.float32),
            pltpu.SemaphoreType.DMA,
            pltpu.SemaphoreType.DMA,
            pltpu.SemaphoreType.DMA,
            pltpu.SemaphoreType.DMA,
            pltpu.SemaphoreType.DMA,
            pltpu.SemaphoreType.DMA,
        ],
    )
    def agg(x_hbm, ei_hbm, ew_hbm, part_hbm,
            src0, src1, dst0, dst1, w0, w1, rows0, rows1,
            dhA0, dhA1, dhB0, dhB1, acc_sh,
            gsem0, gsem1, isem0, isem1, ssem0, ssem1):
        cid = lax.axis_index("c")
        sid = lax.axis_index("s")
        wid = cid * NS + sid
        ebase = wid * EPW

        def fetch_records(t, src_r, dst_r, w_r, sem):
            e0 = ebase + t * CHUNK
            pltpu.async_copy(ei_hbm.at[pl.ds(e0, CHUNK)], src_r, sem)
            pltpu.async_copy(ei_hbm.at[pl.ds(N_EDGES + e0, CHUNK)], dst_r, sem)
            pltpu.async_copy(ew_hbm.at[pl.ds(e0, CHUNK)], w_r, sem)

        def wait_records(t, src_r, dst_r, w_r, sem):
            e0 = ebase + t * CHUNK
            pltpu.make_async_copy(ei_hbm.at[pl.ds(e0, CHUNK)], src_r, sem).wait()
            pltpu.make_async_copy(ei_hbm.at[pl.ds(N_EDGES + e0, CHUNK)], dst_r,
                                  sem).wait()
            pltpu.make_async_copy(ew_hbm.at[pl.ds(e0, CHUNK)], w_r, sem).wait()

        fetch_records(0, src0, dst0, w0, isem0)
        wait_records(0, src0, dst0, w0, isem0)
        pltpu.async_copy(x_hbm.at[src0], rows0, gsem0)
        fetch_records(1, src1, dst1, w1, isem1)

        def zrow(i, c):
            for j in range(D // VEC):
                rows1[i, pl.ds(j * VEC, VEC)] = jnp.zeros((VEC,), jnp.float32)
            return c
        lax.fori_loop(0, CHUNK, zrow, 0)
        base = sid * ROWS_PER_TILE
        for i in range(NFULL):
            pltpu.sync_copy(rows1, acc_sh.at[pl.ds(base + i * CHUNK, CHUNK)])
        pltpu.sync_copy(rows1.at[pl.ds(0, REM)],
                        acc_sh.at[pl.ds(base + NFULL * CHUNK, REM)])

        @pl.when(sid == NS - 1)
        def _zero_tail():
            pltpu.sync_copy(rows1.at[pl.ds(0, TAIL_ROWS)],
                            acc_sh.at[pl.ds(TAIL_ROW0, TAIL_ROWS)])
        plsc.subcore_barrier()

        def drain_scatters(rows_r, dhA_r, dhB_r, ssem_r):
            pltpu.make_async_copy(rows_r.at[pl.ds(0, HALF_A)],
                                  acc_sh.at[dhA_r], ssem_r).wait()
            pltpu.make_async_copy(rows_r.at[pl.ds(HALF_A, HALF_B)],
                                  acc_sh.at[dhB_r], ssem_r).wait()

        def one_iter(t, src_p, dst_p, w_p, rows_p, gsem_p, dhA_p, dhB_p,
                     ssem_p, isem_p,
                     src_q, dst_q, w_q, rows_q, gsem_q, dhA_q, dhB_q,
                     ssem_q, isem_q):
            @pl.when(t + 1 < NCHUNK)
            def _launch_next():
                wait_records(t + 1, src_q, dst_q, w_q, isem_q)

                @pl.when(t >= 1)
                def _drain_prev():
                    drain_scatters(rows_q, dhA_q, dhB_q, ssem_q)
                pltpu.async_copy(x_hbm.at[src_q], rows_q, gsem_q)

            pltpu.make_async_copy(x_hbm.at[src_p], rows_p, gsem_p).wait()

            for h in range(HALF_A // VEC):
                dhA_p[pl.ds(h * VEC, VEC)] = dst_p[pl.ds(h * VEC, VEC)]
            for h in range(HALF_B // VEC):
                dhB_p[pl.ds(h * VEC, VEC)] = dst_p[pl.ds(HALF_A + h * VEC, VEC)]

            def grp_body(g, c):
                w16 = w_p[pl.ds(g * VEC, VEC)]
                for e in range(VEC):
                    wbc = jnp.full((VEC,), w16[e], jnp.float32)
                    row = g * VEC + e
                    for j in range(D // VEC):
                        sl = pl.ds(j * VEC, VEC)
                        rows_p[row, sl] = rows_p[row, sl] * wbc
                return c
            lax.fori_loop(0, HALF_A // VEC, grp_body, 0)
            pltpu.async_copy(rows_p.at[pl.ds(0, HALF_A)],
                             acc_sh.at[dhA_p], ssem_p, add=True)
            lax.fori_loop(HALF_A // VEC, CHUNK // VEC, grp_body, 0)
            pltpu.async_copy(rows_p.at[pl.ds(HALF_A, HALF_B)],
                             acc_sh.at[dhB_p], ssem_p, add=True)

            @pl.when(t + 2 < NCHUNK)
            def _prefetch_rec():
                fetch_records(t + 2, src_p, dst_p, w_p, isem_p)

        def body(t, c):
            @pl.when(lax.rem(t, 2) == 0)
            def _even():
                one_iter(t, src0, dst0, w0, rows0, gsem0, dhA0, dhB0,
                         ssem0, isem0,
                         src1, dst1, w1, rows1, gsem1, dhA1, dhB1,
                         ssem1, isem1)

            @pl.when(lax.rem(t, 2) == 1)
            def _odd():
                one_iter(t, src1, dst1, w1, rows1, gsem1, dhA1, dhB1,
                         ssem1, isem1,
                         src0, dst0, w0, rows0, gsem0, dhA0, dhB0,
                         ssem0, isem0)
            return c
        lax.fori_loop(0, NCHUNK, body, 0)
        drain_scatters(rows1, dhA1, dhB1, ssem1)
        drain_scatters(rows0, dhA0, dhB0, ssem0)
        plsc.subcore_barrier()

        for i in range(NFULL):
            off = base + i * CHUNK
            pltpu.sync_copy(acc_sh.at[pl.ds(off, CHUNK)], rows0)
            pltpu.sync_copy(rows0, part_hbm.at[cid, pl.ds(off, CHUNK)])
        off = base + NFULL * CHUNK
        pltpu.sync_copy(acc_sh.at[pl.ds(off, REM)], rows0.at[pl.ds(0, REM)])
        pltpu.sync_copy(rows0.at[pl.ds(0, REM)], part_hbm.at[cid, pl.ds(off, REM)])

        @pl.when(sid == NS - 1)
        def _dump_tail():
            pltpu.sync_copy(acc_sh.at[pl.ds(TAIL_ROW0, TAIL_ROWS)],
                            rows1.at[pl.ds(0, TAIL_ROWS)])
            pltpu.sync_copy(rows1.at[pl.ds(0, TAIL_ROWS)],
                            part_hbm.at[cid, pl.ds(TAIL_ROW0, TAIL_ROWS)])

    return agg(x, ei_flat, ew)


def _tc_combine_mm(parts, W, b):
    def body(p_ref, w_ref, b_ref, o_ref):
        acc = p_ref[0] + p_ref[1]
        o_ref[...] = jnp.dot(acc, w_ref[...],
                             preferred_element_type=jnp.float32) + b_ref[...]

    BM = 1000
    return pl.pallas_call(
        body,
        grid=(N_NODES // BM,),
        in_specs=[
            pl.BlockSpec((NC, BM, D), lambda i: (0, i, 0)),
            pl.BlockSpec((D, D), lambda i: (0, 0)),
            pl.BlockSpec((1, D), lambda i: (0, 0)),
        ],
        out_specs=pl.BlockSpec((BM, D), lambda i: (i, 0)),
        out_shape=jax.ShapeDtypeStruct((N_NODES, D), jnp.float32),
    )(parts, W, b.reshape(1, D))


def kernel(x, edge_index, edge_weight, W, b):
    ei_flat = edge_index.astype(jnp.int32).reshape(2 * N_EDGES)
    ew = edge_weight.astype(jnp.float32)
    parts = _sc_aggregate(x, ei_flat, ew)
    return _tc_combine_mm(parts, W, b)


# speedup vs baseline: 12.6369x; 1.0856x over previous
import functools

import jax
import jax.numpy as jnp
from jax import lax
from jax.experimental import pallas as pl
from jax.experimental.pallas import tpu as pltpu
from jax.experimental.pallas import tpu_sc as plsc

N_NODES = 10000
N_EDGES = 320000
D = 128
NC = 2
NS = 16
NW = NC * NS
VEC = 16

CHUNK = 80
PAIR = 2 * CHUNK
HALF_A = 48
HALF_B = CHUNK - HALF_A
EPW = N_EDGES // NW
NCHUNK = EPW // CHUNK
ROWS_PER_TILE = 624
TAIL_ROW0 = NS * ROWS_PER_TILE
TAIL_ROWS = N_NODES - TAIL_ROW0
NFULL = ROWS_PER_TILE // CHUNK
REM = ROWS_PER_TILE - NFULL * CHUNK
BLOCKS = tuple([(i * CHUNK, CHUNK) for i in range(NFULL)]
               + [(NFULL * CHUNK, REM)])


def _sc_aggregate(x, ei_flat, ew):
    mesh = plsc.VectorSubcoreMesh(
        core_axis_name="c", subcore_axis_name="s",
        num_cores=NC, num_subcores=NS)

    @functools.partial(
        pl.kernel,
        out_type=jax.ShapeDtypeStruct((NC, N_NODES, D), jnp.float32),
        mesh=mesh,
        scratch_types=[
            pltpu.VMEM((PAIR,), jnp.int32),
            pltpu.VMEM((PAIR,), jnp.int32),
            pltpu.VMEM((PAIR,), jnp.int32),
            pltpu.VMEM((PAIR,), jnp.int32),
            pltpu.VMEM((PAIR,), jnp.float32),
            pltpu.VMEM((PAIR,), jnp.float32),
            pltpu.VMEM((CHUNK, D), jnp.float32),
            pltpu.VMEM((CHUNK, D), jnp.float32),
            pltpu.VMEM((HALF_A,), jnp.int32),
            pltpu.VMEM((HALF_A,), jnp.int32),
            pltpu.VMEM((HALF_B,), jnp.int32),
            pltpu.VMEM((HALF_B,), jnp.int32),
            pltpu.VMEM_SHARED((N_NODES, D), jnp.float32),
            pltpu.SemaphoreType.DMA,
            pltpu.SemaphoreType.DMA,
            pltpu.SemaphoreType.DMA,
            pltpu.SemaphoreType.DMA,
            pltpu.SemaphoreType.DMA,
            pltpu.SemaphoreType.DMA,
        ],
    )
    def agg(x_hbm, ei_hbm, ew_hbm, part_hbm,
            srcP0, srcP1, dstP0, dstP1, wP0, wP1, rows0, rows1,
            dhA0, dhA1, dhB0, dhB1, acc_sh,
            gsem0, gsem1, isem0, isem1, ssem0, ssem1):
        cid = lax.axis_index("c")
        sid = lax.axis_index("s")
        wid = cid * NS + sid
        ebase = wid * EPW

        def pair_off(r):
            return ebase + r * PAIR

        def fetch_pair(r, src_r, dst_r, w_r, sem):
            e0 = pair_off(r)
            pltpu.async_copy(ei_hbm.at[pl.ds(e0, PAIR)], src_r, sem)
            pltpu.async_copy(ei_hbm.at[pl.ds(N_EDGES + e0, PAIR)], dst_r, sem)
            pltpu.async_copy(ew_hbm.at[pl.ds(e0, PAIR)], w_r, sem)

        def wait_pair(r, src_r, dst_r, w_r, sem):
            e0 = pair_off(r)
            pltpu.make_async_copy(ei_hbm.at[pl.ds(e0, PAIR)], src_r, sem).wait()
            pltpu.make_async_copy(ei_hbm.at[pl.ds(N_EDGES + e0, PAIR)], dst_r,
                                  sem).wait()
            pltpu.make_async_copy(ew_hbm.at[pl.ds(e0, PAIR)], w_r, sem).wait()

        fetch_pair(0, srcP0, dstP0, wP0, isem0)
        wait_pair(0, srcP0, dstP0, wP0, isem0)
        pltpu.async_copy(x_hbm.at[srcP0.at[pl.ds(0, CHUNK)]], rows0, gsem0)
        fetch_pair(1, srcP1, dstP1, wP1, isem1)

        def zrow(i, c):
            for j in range(D // VEC):
                rows1[i, pl.ds(j * VEC, VEC)] = jnp.zeros((VEC,), jnp.float32)
            return c
        lax.fori_loop(0, CHUNK, zrow, 0)
        base = sid * ROWS_PER_TILE
        for off_k, sz in BLOCKS:
            pltpu.async_copy(rows1.at[pl.ds(0, sz)],
                             acc_sh.at[pl.ds(base + off_k, sz)], ssem0)

        @pl.when(sid == NS - 1)
        def _zero_tail():
            pltpu.async_copy(rows1.at[pl.ds(0, TAIL_ROWS)],
                             acc_sh.at[pl.ds(TAIL_ROW0, TAIL_ROWS)], ssem0)
        for off_k, sz in BLOCKS:
            pltpu.make_async_copy(rows1.at[pl.ds(0, sz)],
                                  acc_sh.at[pl.ds(base + off_k, sz)],
                                  ssem0).wait()

        @pl.when(sid == NS - 1)
        def _zero_tail_wait():
            pltpu.make_async_copy(rows1.at[pl.ds(0, TAIL_ROWS)],
                                  acc_sh.at[pl.ds(TAIL_ROW0, TAIL_ROWS)],
                                  ssem0).wait()
        plsc.subcore_barrier()

        def drain_scatters(rows_r, dhA_r, dhB_r, ssem_r):
            pltpu.make_async_copy(rows_r.at[pl.ds(0, HALF_A)],
                                  acc_sh.at[dhA_r], ssem_r).wait()
            pltpu.make_async_copy(rows_r.at[pl.ds(HALF_A, HALF_B)],
                                  acc_sh.at[dhB_r], ssem_r).wait()

        def one_iter(t, odd, off,
                     src_c, dst_c, w_c, isem_c,
                     src_n, dst_n, w_n, isem_n,
                     rows_p, gsem_p, dhA_p, dhB_p, ssem_p,
                     rows_q, gsem_q, dhA_q, dhB_q, ssem_q):
            @pl.when(t + 1 < NCHUNK)
            def _launch_next():
                if odd:
                    wait_pair((t + 1) // 2, src_n, dst_n, w_n, isem_n)

                @pl.when(t >= 1)
                def _drain_prev():
                    drain_scatters(rows_q, dhA_q, dhB_q, ssem_q)
                if odd:
                    gidx = src_n.at[pl.ds(0, CHUNK)]
                else:
                    gidx = src_c.at[pl.ds(CHUNK, CHUNK)]
                pltpu.async_copy(x_hbm.at[gidx], rows_q, gsem_q)

            pltpu.make_async_copy(x_hbm.at[src_c.at[pl.ds(off, CHUNK)]],
                                  rows_p, gsem_p).wait()

            for h in range(HALF_A // VEC):
                dhA_p[pl.ds(h * VEC, VEC)] = dst_c[pl.ds(off + h * VEC, VEC)]
            for h in range(HALF_B // VEC):
                dhB_p[pl.ds(h * VEC, VEC)] = dst_c[
                    pl.ds(off + HALF_A + h * VEC, VEC)]

            def grp_body(g, c):
                w16 = w_c[pl.ds(off + g * VEC, VEC)]
                for e in range(VEC):
                    wbc = jnp.full((VEC,), w16[e], jnp.float32)
                    row = g * VEC + e
                    for j in range(D // VEC):
                        sl = pl.ds(j * VEC, VEC)
                        rows_p[row, sl] = rows_p[row, sl] * wbc
                return c
            lax.fori_loop(0, HALF_A // VEC, grp_body, 0)
            pltpu.async_copy(rows_p.at[pl.ds(0, HALF_A)],
                             acc_sh.at[dhA_p], ssem_p, add=True)
            lax.fori_loop(HALF_A // VEC, CHUNK // VEC, grp_body, 0)
            pltpu.async_copy(rows_p.at[pl.ds(HALF_A, HALF_B)],
                             acc_sh.at[dhB_p], ssem_p, add=True)

            if odd:
                @pl.when(t + 3 < NCHUNK)
                def _prefetch_pair():
                    fetch_pair((t + 1) // 2 + 1, src_c, dst_c, w_c, isem_c)

        def body(t, c):
            m = lax.rem(t, 4)

            @pl.when(m == 0)
            def _m0():
                one_iter(t, False, 0,
                         srcP0, dstP0, wP0, isem0,
                         srcP0, dstP0, wP0, isem0,
                         rows0, gsem0, dhA0, dhB0, ssem0,
                         rows1, gsem1, dhA1, dhB1, ssem1)

            @pl.when(m == 1)
            def _m1():
                one_iter(t, True, CHUNK,
                         srcP0, dstP0, wP0, isem0,
                         srcP1, dstP1, wP1, isem1,
                         rows1, gsem1, dhA1, dhB1, ssem1,
                         rows0, gsem0, dhA0, dhB0, ssem0)

            @pl.when(m == 2)
            def _m2():
                one_iter(t, False, 0,
                         srcP1, dstP1, wP1, isem1,
                         srcP1, dstP1, wP1, isem1,
                         rows0, gsem0, dhA0, dhB0, ssem0,
                         rows1, gsem1, dhA1, dhB1, ssem1)

            @pl.when(m == 3)
            def _m3():
                one_iter(t, True, CHUNK,
                         srcP1, dstP1, wP1, isem1,
                         srcP0, dstP0, wP0, isem0,
                         rows1, gsem1, dhA1, dhB1, ssem1,
                         rows0, gsem0, dhA0, dhB0, ssem0)
            return c
        lax.fori_loop(0, NCHUNK, body, 0)
        drain_scatters(rows1, dhA1, dhB1, ssem1)
        drain_scatters(rows0, dhA0, dhB0, ssem0)
        plsc.subcore_barrier()

        bufs = (rows0, rows1)
        insems = (gsem0, gsem1)
        outsems = (ssem0, ssem1)
        off0, sz0 = BLOCKS[0]
        pltpu.async_copy(acc_sh.at[pl.ds(base + off0, sz0)],
                         rows0.at[pl.ds(0, sz0)], gsem0)
        nblk = len(BLOCKS)
        for k, (off_k, sz) in enumerate(BLOCKS):
            buf = bufs[k % 2]
            pltpu.make_async_copy(acc_sh.at[pl.ds(base + off_k, sz)],
                                  buf.at[pl.ds(0, sz)], insems[k % 2]).wait()
            pltpu.async_copy(buf.at[pl.ds(0, sz)],
                             part_hbm.at[cid, pl.ds(base + off_k, sz)],
                             outsems[k % 2])
            if k + 1 < nblk:
                noff, nsz = BLOCKS[k + 1]
                nbuf = bufs[(k + 1) % 2]
                if k >= 1:
                    poff, psz = BLOCKS[k - 1]
                    pltpu.make_async_copy(
                        nbuf.at[pl.ds(0, psz)],
                        part_hbm.at[cid, pl.ds(base + poff, psz)],
                        outsems[(k + 1) % 2]).wait()
                pltpu.async_copy(acc_sh.at[pl.ds(base + noff, nsz)],
                                 nbuf.at[pl.ds(0, nsz)], insems[(k + 1) % 2])
        for k in (nblk - 2, nblk - 1):
            off_k, sz = BLOCKS[k]
            pltpu.make_async_copy(bufs[k % 2].at[pl.ds(0, sz)],
                                  part_hbm.at[cid, pl.ds(base + off_k, sz)],
                                  outsems[k % 2]).wait()

        @pl.when(sid == NS - 1)
        def _dump_tail():
            pltpu.sync_copy(acc_sh.at[pl.ds(TAIL_ROW0, TAIL_ROWS)],
                            rows1.at[pl.ds(0, TAIL_ROWS)])
            pltpu.sync_copy(rows1.at[pl.ds(0, TAIL_ROWS)],
                            part_hbm.at[cid, pl.ds(TAIL_ROW0, TAIL_ROWS)])

    return agg(x, ei_flat, ew)


def _tc_combine_mm(parts, W, b):
    def body(p_ref, w_ref, b_ref, o_ref):
        acc = p_ref[0] + p_ref[1]
        o_ref[...] = jnp.dot(acc, w_ref[...],
                             preferred_element_type=jnp.float32) + b_ref[...]

    BM = 1000
    return pl.pallas_call(
        body,
        grid=(N_NODES // BM,),
        in_specs=[
            pl.BlockSpec((NC, BM, D), lambda i: (0, i, 0)),
            pl.BlockSpec((D, D), lambda i: (0, 0)),
            pl.BlockSpec((1, D), lambda i: (0, 0)),
        ],
        out_specs=pl.BlockSpec((BM, D), lambda i: (i, 0)),
        out_shape=jax.ShapeDtypeStruct((N_NODES, D), jnp.float32),
    )(parts, W, b.reshape(1, D))


def kernel(x, edge_index, edge_weight, W, b):
    ei_flat = jnp.pad(edge_index.astype(jnp.int32).reshape(2 * N_EDGES),
                      (0, CHUNK))
    ew = jnp.pad(edge_weight.astype(jnp.float32), (0, CHUNK))
    parts = _sc_aggregate(x, ei_flat, ew)
    return _tc_combine_mm(parts, W, b)


# speedup vs baseline: 12.8106x; 1.0137x over previous
import functools

import jax
import jax.numpy as jnp
from jax import lax
from jax.experimental import pallas as pl
from jax.experimental.pallas import tpu as pltpu
from jax.experimental.pallas import tpu_sc as plsc

N_NODES = 10000
N_EDGES = 320000
D = 128
NC = 2
NS = 16
NW = NC * NS
VEC = 16

CHUNK = 80
PAIR = 2 * CHUNK
HALF_A = 48
HALF_B = CHUNK - HALF_A
EPW = N_EDGES // NW
NCHUNK = EPW // CHUNK
ROWS_PER_TILE = 624
TAIL_ROW0 = NS * ROWS_PER_TILE
TAIL_ROWS = N_NODES - TAIL_ROW0
NFULL = ROWS_PER_TILE // CHUNK
REM = ROWS_PER_TILE - NFULL * CHUNK
BLOCKS = tuple([(i * CHUNK, CHUNK) for i in range(NFULL)]
               + [(NFULL * CHUNK, REM)])


def _sc_aggregate(x, ei_flat, ew):
    mesh = plsc.VectorSubcoreMesh(
        core_axis_name="c", subcore_axis_name="s",
        num_cores=NC, num_subcores=NS)

    @functools.partial(
        pl.kernel,
        out_type=jax.ShapeDtypeStruct((NC, N_NODES, D), jnp.float32),
        mesh=mesh,
        scratch_types=[
            pltpu.VMEM((PAIR,), jnp.int32),
            pltpu.VMEM((PAIR,), jnp.int32),
            pltpu.VMEM((PAIR,), jnp.int32),
            pltpu.VMEM((PAIR,), jnp.int32),
            pltpu.VMEM((PAIR,), jnp.float32),
            pltpu.VMEM((PAIR,), jnp.float32),
            pltpu.VMEM((CHUNK, D), jnp.float32),
            pltpu.VMEM((CHUNK, D), jnp.float32),
            pltpu.VMEM((HALF_A,), jnp.int32),
            pltpu.VMEM((HALF_A,), jnp.int32),
            pltpu.VMEM((HALF_B,), jnp.int32),
            pltpu.VMEM((HALF_B,), jnp.int32),
            pltpu.VMEM_SHARED((N_NODES, D), jnp.float32),
            pltpu.SemaphoreType.DMA,
            pltpu.SemaphoreType.DMA,
            pltpu.SemaphoreType.DMA,
            pltpu.SemaphoreType.DMA,
            pltpu.SemaphoreType.DMA,
            pltpu.SemaphoreType.DMA,
        ],
    )
    def agg(x_hbm, ei_hbm, ew_hbm, part_hbm,
            srcP0, srcP1, dstP0, dstP1, wP0, wP1, rows0, rows1,
            dhA0, dhA1, dhB0, dhB1, acc_sh,
            gsem0, gsem1, isem0, isem1, ssem0, ssem1):
        cid = lax.axis_index("c")
        sid = lax.axis_index("s")
        wid = cid * NS + sid
        ebase = wid * EPW

        def pair_off(r):
            return ebase + r * PAIR

        def fetch_pair(r, src_r, dst_r, w_r, sem):
            e0 = pair_off(r)
            pltpu.async_copy(ei_hbm.at[pl.ds(e0, PAIR)], src_r, sem)
            pltpu.async_copy(ei_hbm.at[pl.ds(N_EDGES + e0, PAIR)], dst_r, sem)
            pltpu.async_copy(ew_hbm.at[pl.ds(e0, PAIR)], w_r, sem)

        def wait_pair(r, src_r, dst_r, w_r, sem):
            e0 = pair_off(r)
            pltpu.make_async_copy(ei_hbm.at[pl.ds(e0, PAIR)], src_r, sem).wait()
            pltpu.make_async_copy(ei_hbm.at[pl.ds(N_EDGES + e0, PAIR)], dst_r,
                                  sem).wait()
            pltpu.make_async_copy(ew_hbm.at[pl.ds(e0, PAIR)], w_r, sem).wait()

        fetch_pair(0, srcP0, dstP0, wP0, isem0)
        wait_pair(0, srcP0, dstP0, wP0, isem0)
        pltpu.async_copy(x_hbm.at[srcP0.at[pl.ds(0, CHUNK)]], rows0, gsem0)
        fetch_pair(1, srcP1, dstP1, wP1, isem1)

        def zrow(i, c):
            for j in range(D // VEC):
                rows1[i, pl.ds(j * VEC, VEC)] = jnp.zeros((VEC,), jnp.float32)
            return c
        lax.fori_loop(0, CHUNK, zrow, 0)
        base = sid * ROWS_PER_TILE
        for off_k, sz in BLOCKS:
            pltpu.async_copy(rows1.at[pl.ds(0, sz)],
                             acc_sh.at[pl.ds(base + off_k, sz)], ssem0)

        @pl.when(sid == NS - 1)
        def _zero_tail():
            pltpu.async_copy(rows1.at[pl.ds(0, TAIL_ROWS)],
                             acc_sh.at[pl.ds(TAIL_ROW0, TAIL_ROWS)], ssem0)
        for off_k, sz in BLOCKS:
            pltpu.make_async_copy(rows1.at[pl.ds(0, sz)],
                                  acc_sh.at[pl.ds(base + off_k, sz)],
                                  ssem0).wait()

        @pl.when(sid == NS - 1)
        def _zero_tail_wait():
            pltpu.make_async_copy(rows1.at[pl.ds(0, TAIL_ROWS)],
                                  acc_sh.at[pl.ds(TAIL_ROW0, TAIL_ROWS)],
                                  ssem0).wait()
        plsc.subcore_barrier()

        def drain_scatters(rows_r, dhA_r, dhB_r, ssem_r):
            pltpu.make_async_copy(rows_r.at[pl.ds(0, HALF_A)],
                                  acc_sh.at[dhA_r], ssem_r).wait()
            pltpu.make_async_copy(rows_r.at[pl.ds(HALF_A, HALF_B)],
                                  acc_sh.at[dhB_r], ssem_r).wait()

        def one_iter(t, odd, off,
                     src_c, dst_c, w_c, isem_c,
                     src_n, dst_n, w_n, isem_n,
                     rows_p, gsem_p, dhA_p, dhB_p, ssem_p,
                     rows_q, gsem_q, dhA_q, dhB_q, ssem_q):
            @pl.when(t + 1 < NCHUNK)
            def _launch_next():
                if odd:
                    wait_pair((t + 1) // 2, src_n, dst_n, w_n, isem_n)

                @pl.when(t >= 1)
                def _drain_prev():
                    drain_scatters(rows_q, dhA_q, dhB_q, ssem_q)
                if odd:
                    gidx = src_n.at[pl.ds(0, CHUNK)]
                else:
                    gidx = src_c.at[pl.ds(CHUNK, CHUNK)]
                pltpu.async_copy(x_hbm.at[gidx], rows_q, gsem_q)

            pltpu.make_async_copy(x_hbm.at[src_c.at[pl.ds(off, CHUNK)]],
                                  rows_p, gsem_p).wait()

            for h in range(HALF_A // VEC):
                dhA_p[pl.ds(h * VEC, VEC)] = dst_c[pl.ds(off + h * VEC, VEC)]
            for h in range(HALF_B // VEC):
                dhB_p[pl.ds(h * VEC, VEC)] = dst_c[
                    pl.ds(off + HALF_A + h * VEC, VEC)]

            def grp_body(g, c):
                w16 = w_c[pl.ds(off + g * VEC, VEC)]
                for e in range(VEC):
                    wbc = jnp.full((VEC,), w16[e], jnp.float32)
                    row = g * VEC + e
                    for j in range(D // VEC):
                        sl = pl.ds(j * VEC, VEC)
                        rows_p[row, sl] = rows_p[row, sl] * wbc
                return c
            lax.fori_loop(0, HALF_A // VEC, grp_body, 0)
            pltpu.async_copy(rows_p.at[pl.ds(0, HALF_A)],
                             acc_sh.at[dhA_p], ssem_p, add=True)
            lax.fori_loop(HALF_A // VEC, CHUNK // VEC, grp_body, 0)
            pltpu.async_copy(rows_p.at[pl.ds(HALF_A, HALF_B)],
                             acc_sh.at[dhB_p], ssem_p, add=True)

            if odd:
                @pl.when(t + 3 < NCHUNK)
                def _prefetch_pair():
                    fetch_pair((t + 1) // 2 + 1, src_c, dst_c, w_c, isem_c)

        def body(t, c):
            m = lax.rem(t, 4)

            @pl.when(m == 0)
            def _m0():
                one_iter(t, False, 0,
                         srcP0, dstP0, wP0, isem0,
                         srcP0, dstP0, wP0, isem0,
                         rows0, gsem0, dhA0, dhB0, ssem0,
                         rows1, gsem1, dhA1, dhB1, ssem1)

            @pl.when(m == 1)
            def _m1():
                one_iter(t, True, CHUNK,
                         srcP0, dstP0, wP0, isem0,
                         srcP1, dstP1, wP1, isem1,
                         rows1, gsem1, dhA1, dhB1, ssem1,
                         rows0, gsem0, dhA0, dhB0, ssem0)

            @pl.when(m == 2)
            def _m2():
                one_iter(t, False, 0,
                         srcP1, dstP1, wP1, isem1,
                         srcP1, dstP1, wP1, isem1,
                         rows0, gsem0, dhA0, dhB0, ssem0,
                         rows1, gsem1, dhA1, dhB1, ssem1)

            @pl.when(m == 3)
            def _m3():
                one_iter(t, True, CHUNK,
                         srcP1, dstP1, wP1, isem1,
                         srcP0, dstP0, wP0, isem0,
                         rows1, gsem1, dhA1, dhB1, ssem1,
                         rows0, gsem0, dhA0, dhB0, ssem0)
            return c
        lax.fori_loop(0, NCHUNK, body, 0)
        drain_scatters(rows1, dhA1, dhB1, ssem1)
        drain_scatters(rows0, dhA0, dhB0, ssem0)
        plsc.subcore_barrier()

        bufs = (rows0, rows1)
        insems = (gsem0, gsem1)
        outsems = (ssem0, ssem1)
        off0, sz0 = BLOCKS[0]
        pltpu.async_copy(acc_sh.at[pl.ds(base + off0, sz0)],
                         rows0.at[pl.ds(0, sz0)], gsem0)
        nblk = len(BLOCKS)
        for k, (off_k, sz) in enumerate(BLOCKS):
            buf = bufs[k % 2]
            pltpu.make_async_copy(acc_sh.at[pl.ds(base + off_k, sz)],
                                  buf.at[pl.ds(0, sz)], insems[k % 2]).wait()
            pltpu.async_copy(buf.at[pl.ds(0, sz)],
                             part_hbm.at[cid, pl.ds(base + off_k, sz)],
                             outsems[k % 2])
            if k + 1 < nblk:
                noff, nsz = BLOCKS[k + 1]
                nbuf = bufs[(k + 1) % 2]
                if k >= 1:
                    poff, psz = BLOCKS[k - 1]
                    pltpu.make_async_copy(
                        nbuf.at[pl.ds(0, psz)],
                        part_hbm.at[cid, pl.ds(base + poff, psz)],
                        outsems[(k + 1) % 2]).wait()
                pltpu.async_copy(acc_sh.at[pl.ds(base + noff, nsz)],
                                 nbuf.at[pl.ds(0, nsz)], insems[(k + 1) % 2])
        for k in (nblk - 2, nblk - 1):
            off_k, sz = BLOCKS[k]
            pltpu.make_async_copy(bufs[k % 2].at[pl.ds(0, sz)],
                                  part_hbm.at[cid, pl.ds(base + off_k, sz)],
                                  outsems[k % 2]).wait()

        @pl.when(sid == NS - 1)
        def _dump_tail():
            pltpu.sync_copy(acc_sh.at[pl.ds(TAIL_ROW0, TAIL_ROWS)],
                            rows1.at[pl.ds(0, TAIL_ROWS)])
            pltpu.sync_copy(rows1.at[pl.ds(0, TAIL_ROWS)],
                            part_hbm.at[cid, pl.ds(TAIL_ROW0, TAIL_ROWS)])

    return agg(x, ei_flat, ew)


def _tc_combine_mm(parts, W, b):
    def body(p_ref, w_ref, b_ref, o_ref):
        acc = p_ref[0] + p_ref[1]
        o_ref[...] = jnp.dot(acc, w_ref[...],
                             preferred_element_type=jnp.float32) + b_ref[...]

    BM = 2000
    return pl.pallas_call(
        body,
        grid=(N_NODES // BM,),
        in_specs=[
            pl.BlockSpec((NC, BM, D), lambda i: (0, i, 0)),
            pl.BlockSpec((D, D), lambda i: (0, 0)),
            pl.BlockSpec((1, D), lambda i: (0, 0)),
        ],
        out_specs=pl.BlockSpec((BM, D), lambda i: (i, 0)),
        out_shape=jax.ShapeDtypeStruct((N_NODES, D), jnp.float32),
    )(parts, W, b.reshape(1, D))


def kernel(x, edge_index, edge_weight, W, b):
    ei_flat = jnp.pad(edge_index.astype(jnp.int32).reshape(2 * N_EDGES),
                      (0, CHUNK))
    ew = jnp.pad(edge_weight.astype(jnp.float32), (0, CHUNK))
    parts = _sc_aggregate(x, ei_flat, ew)
    return _tc_combine_mm(parts, W, b)
